# baseline (speedup 1.0000x reference)
import jax
import jax.numpy as jnp
from jax.experimental import pallas as pl
from jax.experimental.pallas import tpu as pltpu

_TM = 512
_CHUNKS = ((0, 256), (256, 256), (512, 256), (768, 232))


def _w_copy(w_hbm, w_vmem, sems, c):
    lo, sz = _CHUNKS[c]
    return pltpu.make_async_copy(
        w_hbm.at[pl.ds(lo, sz), :],
        w_vmem.at[pl.ds(lo, sz), :],
        sems.at[c],
    )


def _linear_kernel(x_ref, w_hbm, b_ref, o_ref, w_vmem, sems):
    i = pl.program_id(0)

    @pl.when(i == 0)
    def _():
        for c in range(len(_CHUNKS)):
            _w_copy(w_hbm, w_vmem, sems, c).start()

    x = x_ref[...]
    for c, (lo, sz) in enumerate(_CHUNKS):
        @pl.when(i == 0)
        def _(c=c):
            _w_copy(w_hbm, w_vmem, sems, c).wait()

        o_ref[:, lo:lo + sz] = (
            jax.lax.dot_general(
                x, w_vmem[pl.ds(lo, sz), :],
                dimension_numbers=(((1,), (1,)), ((), ())),
                preferred_element_type=jnp.float32,
            )
            + b_ref[:, lo:lo + sz]
        )


@jax.jit
def _forward(x, weight, bias):
    B = x.shape[0]
    F = x.shape[1] * x.shape[2] * x.shape[3]
    O = weight.shape[0]

    x_flat = x.reshape(B, F)
    b2 = bias.reshape(1, O)
    grid_m = B // _TM

    return pl.pallas_call(
        _linear_kernel,
        out_shape=jax.ShapeDtypeStruct((B, O), jnp.float32),
        grid=(grid_m,),
        in_specs=[
            pl.BlockSpec((_TM, F), lambda i: (i, 0)),
            pl.BlockSpec(memory_space=pl.ANY),
            pl.BlockSpec((1, O), lambda i: (0, 0)),
        ],
        out_specs=pl.BlockSpec((_TM, O), lambda i: (i, 0)),
        scratch_shapes=[
            pltpu.VMEM((O, F), jnp.float32),
            pltpu.SemaphoreType.DMA((len(_CHUNKS),)),
        ],
        compiler_params=pltpu.CompilerParams(
            dimension_semantics=("arbitrary",),
            vmem_limit_bytes=40 << 20,
        ),
    )(x_flat, weight, b2)


def kernel(x, weight, bias):
    return _forward(x, weight, bias)


# speedup vs baseline: 1.2756x; 1.2756x over previous
import jax
import jax.numpy as jnp
from jax.experimental import pallas as pl
from jax.experimental.pallas import tpu as pltpu

_TM = 512


def _linear_kernel(x_ref, w_ref, b_ref, o_ref):
    o_ref[...] = (
        jax.lax.dot_general(
            x_ref[...], w_ref[...],
            dimension_numbers=(((1,), (1,)), ((), ())),
            preferred_element_type=jnp.float32,
        )
        + b_ref[...]
    )


@jax.jit
def _forward(x, weight, bias):
    B = x.shape[0]
    F = x.shape[1] * x.shape[2] * x.shape[3]
    O = weight.shape[0]

    x_flat = x.reshape(B, F)
    b2 = bias.reshape(1, O)
    grid_m = B // _TM

    return pl.pallas_call(
        _linear_kernel,
        out_shape=jax.ShapeDtypeStruct((B, O), jnp.float32),
        grid=(grid_m,),
        in_specs=[
            pl.BlockSpec((_TM, F), lambda i: (i, 0)),
            pl.BlockSpec((O, F), lambda i: (0, 0)),
            pl.BlockSpec((1, O), lambda i: (0, 0)),
        ],
        out_specs=pl.BlockSpec((_TM, O), lambda i: (i, 0)),
        compiler_params=pltpu.CompilerParams(
            dimension_semantics=("arbitrary",),
            vmem_limit_bytes=40 << 20,
        ),
    )(x_flat, weight, b2)


def kernel(x, weight, bias):
    return _forward(x, weight, bias)
